# Initial kernel scaffold; baseline (speedup 1.0000x reference)
import jax
import jax.numpy as jnp
from jax.experimental import pallas as pl


def kernel(x, edge_index, batch, W1, b1, W2, b2, W3, b3, M1, mb1, M2, mb2, M3, mb3):
    raise NotImplementedError("write your pallas kernel here")


# speedup vs baseline: 17.2763x; 17.2763x over previous
import jax
import jax.numpy as jnp
from jax import lax
from jax.experimental import pallas as pl
from jax.experimental.pallas import tpu as pltpu
from jax.experimental.pallas import tpu_sc as plsc

N = 10000
E = 320000
D = 128
H = 128
C = 16
G = 256

NC = 2
NS = 16
NW = NC * NS
CH = 128
NCHUNK = 80
EPW = NCHUNK * CH
EP = NW * EPW
NP = 10240
GP = 384
RPS = NP // NS
R = 1024


def _mesh():
    return plsc.VectorSubcoreMesh(core_axis_name="c", subcore_axis_name="s",
                                  num_cores=NC, num_subcores=NS)


def _worker_ids():
    c = lax.axis_index("c")
    s = lax.axis_index("s")
    return c, s, c * NS + s


def _fill(ref, rows, value):
    width = ref.shape[-1]
    vec = jnp.full((16,), value, dtype=ref.dtype)

    def body(i):
        for k in range(width // 16):
            ref[i, pl.ds(k * 16, 16)] = vec

    lax.fori_loop(0, rows, lambda i, _: (body(i), None)[1], None)


def _prop_body(p_hbm, src_hbm, dst_hbm, out_hbm, si0, di0, si1, di1, r0, r1,
               acc, sem0, sem1):
    c, s, w = _worker_ids()
    _fill(r0, CH, 0.0)
    for r in range(RPS // CH):
        pltpu.sync_copy(r0, acc.at[pl.ds(s * RPS + r * CH, CH)])
    plsc.subcore_barrier()

    pltpu.sync_copy(src_hbm.at[w, 0], si0)
    pltpu.sync_copy(dst_hbm.at[w, 0], di0)
    pltpu.async_copy(p_hbm.at[si0], r0, sem0)

    @pl.loop(0, NCHUNK // 2)
    def _pairs(t):
        j = 2 * t
        pltpu.sync_copy(src_hbm.at[w, j + 1], si1)
        pltpu.sync_copy(dst_hbm.at[w, j + 1], di1)
        pltpu.async_copy(p_hbm.at[si1], r1, sem1)
        pltpu.make_async_copy(p_hbm.at[si0], r0, sem0).wait()
        pltpu.sync_copy(r0, acc.at[di0], add=True)

        @pl.when(j + 2 < NCHUNK)
        def _():
            pltpu.sync_copy(src_hbm.at[w, j + 2], si0)
            pltpu.sync_copy(dst_hbm.at[w, j + 2], di0)
            pltpu.async_copy(p_hbm.at[si0], r0, sem0)

        pltpu.make_async_copy(p_hbm.at[si1], r1, sem1).wait()
        pltpu.sync_copy(r1, acc.at[di1], add=True)

    plsc.subcore_barrier()
    pltpu.sync_copy(acc.at[pl.ds(s * RPS, RPS)],
                    out_hbm.at[pl.ds(c * NP + s * RPS, RPS)])


def _prop_call(p, src3, dst3):
    kern = pl.kernel(
        _prop_body,
        out_type=jax.ShapeDtypeStruct((NC * NP, H), jnp.float32),
        mesh=_mesh(),
        scratch_types=[
            pltpu.VMEM((CH,), jnp.int32),
            pltpu.VMEM((CH,), jnp.int32),
            pltpu.VMEM((CH,), jnp.int32),
            pltpu.VMEM((CH,), jnp.int32),
            pltpu.VMEM((CH, H), jnp.float32),
            pltpu.VMEM((CH, H), jnp.float32),
            pltpu.VMEM_SHARED((NP, H), jnp.float32),
            pltpu.SemaphoreType.DMA,
            pltpu.SemaphoreType.DMA,
        ],
    )
    return kern(p, src3, dst3)


def _pool_body(h_hbm, batch_hbm, out_hbm, batch_idx, rows_v, acc, sem):
    c, s, w = _worker_ids()
    _fill(rows_v, CH, 0.0)
    pltpu.sync_copy(rows_v.at[pl.ds(0, GP // NS)],
                    acc.at[pl.ds(s * (GP // NS), GP // NS)])
    plsc.subcore_barrier()

    qb = jnp.where(w < 16, 3 * w, 48 + 2 * (w - 16))

    def chunk(q):
        pltpu.sync_copy(batch_hbm.at[q], batch_idx)
        pltpu.async_copy(h_hbm.at[pl.ds(q * CH, CH)], rows_v, sem).wait()
        pltpu.sync_copy(rows_v, acc.at[batch_idx], add=True)

    chunk(qb)
    chunk(qb + 1)

    @pl.when(w < 16)
    def _():
        chunk(qb + 2)

    plsc.subcore_barrier()
    pltpu.sync_copy(acc.at[pl.ds(s * (GP // NS), GP // NS)],
                    out_hbm.at[pl.ds(c * GP + s * (GP // NS), GP // NS)])


def _pool_call(h, batch_q):
    kern = pl.kernel(
        _pool_body,
        out_type=jax.ShapeDtypeStruct((NC * GP, H), jnp.float32),
        mesh=_mesh(),
        scratch_types=[
            pltpu.VMEM((CH,), jnp.int32),
            pltpu.VMEM((CH, H), jnp.float32),
            pltpu.VMEM_SHARED((GP, H), jnp.float32),
            pltpu.SemaphoreType.DMA,
        ],
    )
    return kern(h, batch_q)


def _dinv_block(deg_ref):
    d = deg_ref[0][:, 0:1] + deg_ref[1][:, 0:1] + 1.0
    return lax.rsqrt(jnp.maximum(d, 1.0))


def _mm_scale_body(x_ref, w_ref, deg_ref, o_ref):
    dinv = _dinv_block(deg_ref)
    o_ref[...] = jnp.dot(x_ref[...], w_ref[...],
                         preferred_element_type=jnp.float32) * dinv


def _layer_body(acc_ref, p_ref, deg_ref, b_ref, w_ref, o_ref):
    dinv = _dinv_block(deg_ref)
    h = jnp.maximum(dinv * (acc_ref[0] + acc_ref[1] + p_ref[...]) + b_ref[...],
                    0.0)
    o_ref[...] = jnp.dot(h, w_ref[...],
                         preferred_element_type=jnp.float32) * dinv


def _relu_body(acc_ref, p_ref, deg_ref, b_ref, o_ref):
    dinv = _dinv_block(deg_ref)
    o_ref[...] = jnp.maximum(
        dinv * (acc_ref[0] + acc_ref[1] + p_ref[...]) + b_ref[...], 0.0)


def _head_body(g_ref, m1_ref, b1_ref, m2_ref, b2_ref, m3_ref, b3_ref, o_ref):
    g = g_ref[0, 0:G, :] + g_ref[1, 0:G, :]
    a = jnp.maximum(jnp.dot(g, m1_ref[...],
                            preferred_element_type=jnp.float32) + b1_ref[...],
                    0.0)
    a = jnp.maximum(jnp.dot(a, m2_ref[...],
                            preferred_element_type=jnp.float32) + b2_ref[...],
                    0.0)
    o_ref[...] = jnp.dot(a, m3_ref[...],
                         preferred_element_type=jnp.float32) + b3_ref[...]


_row_spec = pl.BlockSpec((R, H), lambda i: (i, 0))
_acc_spec = pl.BlockSpec((2, R, H), lambda i: (0, i, 0))
_deg_spec = pl.BlockSpec((2, R, H), lambda i: (0, i, 0))
_w_spec = pl.BlockSpec((H, H), lambda i: (0, 0))
_b_spec = pl.BlockSpec((1, H), lambda i: (0, 0))


def _mm_scale(x_p, w, deg):
    return pl.pallas_call(
        _mm_scale_body,
        grid=(NP // R,),
        in_specs=[_row_spec, _w_spec, _deg_spec],
        out_specs=_row_spec,
        out_shape=jax.ShapeDtypeStruct((NP, H), jnp.float32),
    )(x_p, w, deg)


def _layer(acc, p, deg, b, w):
    return pl.pallas_call(
        _layer_body,
        grid=(NP // R,),
        in_specs=[_acc_spec, _row_spec, _deg_spec, _b_spec, _w_spec],
        out_specs=_row_spec,
        out_shape=jax.ShapeDtypeStruct((NP, H), jnp.float32),
    )(acc, p, deg, b, w)


def _relu_comb(acc, p, deg, b):
    return pl.pallas_call(
        _relu_body,
        grid=(NP // R,),
        in_specs=[_acc_spec, _row_spec, _deg_spec, _b_spec],
        out_specs=_row_spec,
        out_shape=jax.ShapeDtypeStruct((NP, H), jnp.float32),
    )(acc, p, deg, b)


def _head(g, m1, b1, m2, b2, m3, b3):
    return pl.pallas_call(
        _head_body,
        in_specs=[pl.BlockSpec((2, GP, H), lambda: (0, 0, 0)),
                  pl.BlockSpec((H, H), lambda: (0, 0)),
                  pl.BlockSpec((1, H), lambda: (0, 0)),
                  pl.BlockSpec((H, H), lambda: (0, 0)),
                  pl.BlockSpec((1, H), lambda: (0, 0)),
                  pl.BlockSpec((H, H), lambda: (0, 0)),
                  pl.BlockSpec((1, H), lambda: (0, 0))],
        out_specs=pl.BlockSpec((G, H), lambda: (0, 0)),
        out_shape=jax.ShapeDtypeStruct((G, H), jnp.float32),
    )(g, m1, b1, m2, b2, m3, b3)


def kernel(x, edge_index, batch, W1, b1, W2, b2, W3, b3, M1, mb1, M2, mb2,
           M3, mb3):
    f32 = jnp.float32
    x_p = jnp.zeros((NP, D), f32).at[:N].set(x)
    ones_t = jnp.ones((NP, D), f32)
    pad_e = EP - E
    epad = (N + (jnp.arange(pad_e, dtype=jnp.int32) % (NP - N))).astype(
        jnp.int32)
    src3 = jnp.concatenate([edge_index[0], epad]).reshape(NW, NCHUNK, CH)
    dst3 = jnp.concatenate([edge_index[1], epad]).reshape(NW, NCHUNK, CH)
    bpad = (G + (jnp.arange(NP - N, dtype=jnp.int32) % (GP - G))).astype(
        jnp.int32)
    batch_q = jnp.concatenate([batch, bpad]).reshape(NP // CH, CH)

    b1r = b1.reshape(1, H)
    b2r = b2.reshape(1, H)
    b3r = b3.reshape(1, H)
    mb1r = mb1.reshape(1, H)
    mb2r = mb2.reshape(1, H)
    m3p = jnp.zeros((H, H), f32).at[:, :C].set(M3)
    mb3p = jnp.zeros((1, H), f32).at[0, :C].set(mb3)

    deg = _prop_call(ones_t, src3, dst3).reshape(NC, NP, H)
    p = _mm_scale(x_p, W1, deg)
    acc = _prop_call(p, src3, dst3).reshape(NC, NP, H)
    p = _layer(acc, p, deg, b1r, W2)
    acc = _prop_call(p, src3, dst3).reshape(NC, NP, H)
    p = _layer(acc, p, deg, b2r, W3)
    acc = _prop_call(p, src3, dst3).reshape(NC, NP, H)
    h3 = _relu_comb(acc, p, deg, b3r)
    g = _pool_call(h3, batch_q).reshape(NC, GP, H)
    out = _head(g, M1, mb1r, M2, mb2r, m3p, mb3p)
    return out[:, :C]
